# Initial kernel scaffold; baseline (speedup 1.0000x reference)
import jax
import jax.numpy as jnp
from jax.experimental import pallas as pl


def kernel(x, params, edge_index, batch):
    raise NotImplementedError("write your pallas kernel here")


# speedup vs baseline: 7.1992x; 7.1992x over previous
import functools

import jax
import jax.numpy as jnp
from jax import lax
from jax.experimental import pallas as pl
from jax.experimental.pallas import tpu as pltpu
from jax.experimental.pallas import tpu_sc as plsc

N = 10000
E = 320000
F = 128
H = 32
G = 64
CLS = 10

NC = 2
NS = 16
NW = NC * NS
CH = 128
K = 80
EPAD = NW * K * CH
NPAD = 10240
RZ = NPAD // NS


def _segsum_body(p_hbm, src_hbm, dst_hbm, zeros_hbm, out_hbm, srcb, dstb, rows, acc, sem):
    c = lax.axis_index("c")
    s = lax.axis_index("s")
    wid = c * NS + s
    pltpu.sync_copy(zeros_hbm.at[pl.ds(s * RZ, RZ)], acc.at[pl.ds(s * RZ, RZ)])
    pltpu.sync_copy(src_hbm.at[pl.ds(wid * K, K)], srcb)
    pltpu.sync_copy(dst_hbm.at[pl.ds(wid * K, K)], dstb)
    plsc.subcore_barrier()

    def step(j, carry):
        pltpu.async_copy(p_hbm.at[srcb.at[j]], rows, sem).wait()
        pltpu.sync_copy(rows, acc.at[dstb.at[j]], add=True)
        return carry

    lax.fori_loop(0, K, step, 0)
    plsc.subcore_barrier()
    pltpu.sync_copy(acc.at[pl.ds(s * RZ, RZ)], out_hbm.at[c, pl.ds(s * RZ, RZ)])


@functools.lru_cache(maxsize=1)
def _make_segsum():
    return pl.kernel(
        _segsum_body,
        out_type=jax.ShapeDtypeStruct((NC, NPAD, H), jnp.float32),
        mesh=plsc.VectorSubcoreMesh(core_axis_name="c", subcore_axis_name="s"),
        scratch_types=[
            pltpu.VMEM((K, CH), jnp.int32),
            pltpu.VMEM((K, CH), jnp.int32),
            pltpu.VMEM((CH, H), jnp.float32),
            pltpu.VMEM_SHARED((NPAD, H), jnp.float32),
            pltpu.SemaphoreType.DMA,
        ],
        compiler_params=pltpu.CompilerParams(use_tc_tiling_on_sc=False),
    )


def _proj_body(x_ref, w_ref, o_ref):
    o_ref[...] = jnp.dot(x_ref[...], w_ref[...], preferred_element_type=jnp.float32)


_proj = pl.pallas_call(_proj_body, out_shape=jax.ShapeDtypeStruct((N, H), jnp.float32))


def _mlp_bn(p, pa, pb, b1, w2, b2, gam, bet):
    z = jnp.maximum(p + pa + pb + b1, 0.0)
    z = jnp.maximum(jnp.dot(z, w2, preferred_element_type=jnp.float32) + b2, 0.0)
    mu = jnp.mean(z, axis=0, keepdims=True)
    zc = z - mu
    var = jnp.mean(zc * zc, axis=0, keepdims=True)
    return zc * lax.rsqrt(var + 1e-5) * gam + bet


def _layer_body(p_ref, pa_ref, pb_ref, b1_ref, w2_ref, b2_ref, g_ref, be_ref,
                w1n_ref, o_ref):
    h = _mlp_bn(p_ref[...], pa_ref[...], pb_ref[...], b1_ref[...], w2_ref[...],
                b2_ref[...], g_ref[...], be_ref[...])
    o_ref[...] = jnp.dot(h, w1n_ref[...], preferred_element_type=jnp.float32)


_layer = pl.pallas_call(_layer_body, out_shape=jax.ShapeDtypeStruct((N, H), jnp.float32))


def _head_body(p_ref, pa_ref, pb_ref, b1_ref, w2_ref, b2_ref, g_ref, be_ref,
               batch_ref, fc1w_ref, fc1b_ref, fc2w_ref, fc2b_ref, o_ref):
    h = _mlp_bn(p_ref[...], pa_ref[...], pb_ref[...], b1_ref[...], w2_ref[...],
                b2_ref[...], g_ref[...], be_ref[...])
    bo = (batch_ref[...] == lax.broadcasted_iota(jnp.int32, (1, G), 1))
    bo = bo.astype(jnp.float32)
    dn = (((0,), (0,)), ((), ()))
    sums = lax.dot_general(bo, h, dn, preferred_element_type=jnp.float32)
    counts = lax.dot_general(bo, jnp.ones((N, 1), jnp.float32), dn,
                             preferred_element_type=jnp.float32)
    pooled = sums / jnp.maximum(counts, 1.0)
    z = jnp.maximum(jnp.dot(pooled, fc1w_ref[...],
                            preferred_element_type=jnp.float32) + fc1b_ref[...], 0.0)
    logits = jnp.dot(z, fc2w_ref[...], preferred_element_type=jnp.float32) + fc2b_ref[...]
    m = jnp.max(logits, axis=-1, keepdims=True)
    lse = m + jnp.log(jnp.sum(jnp.exp(logits - m), axis=-1, keepdims=True))
    o_ref[...] = logits - lse


_head = pl.pallas_call(_head_body, out_shape=jax.ShapeDtypeStruct((G, CLS), jnp.float32))


def kernel(x, params, edge_index, batch):
    ei = edge_index.astype(jnp.int32)
    pad = EPAD - E
    src2 = jnp.concatenate([ei[0], jnp.zeros((pad,), jnp.int32)]).reshape(NW * K, CH)
    dst2 = jnp.concatenate([ei[1], jnp.full((pad,), N, jnp.int32)]).reshape(NW * K, CH)
    zeros = jnp.zeros((NPAD, H), jnp.float32)
    b2d = batch.astype(jnp.int32).reshape(N, 1)
    vec = lambda v: v.reshape(1, -1)

    segsum = _make_segsum()
    p = _proj(x, params["conv1_W1"])
    for i in range(1, 6):
        parts = segsum(p, src2, dst2, zeros)
        pa = parts[0, :N]
        pb = parts[1, :N]
        if i < 5:
            p = _layer(p, pa, pb, vec(params[f"conv{i}_b1"]), params[f"conv{i}_W2"],
                       vec(params[f"conv{i}_b2"]), vec(params[f"bn{i}_gamma"]),
                       vec(params[f"bn{i}_beta"]), params[f"conv{i + 1}_W1"])
        else:
            out = _head(p, pa, pb, vec(params[f"conv{i}_b1"]), params[f"conv{i}_W2"],
                        vec(params[f"conv{i}_b2"]), vec(params[f"bn{i}_gamma"]),
                        vec(params[f"bn{i}_beta"]), b2d, params["fc1_W"],
                        vec(params["fc1_b"]), params["fc2_W"], vec(params["fc2_b"]))
    return out


# speedup vs baseline: 9.1200x; 1.2668x over previous
import functools

import jax
import jax.numpy as jnp
from jax import lax
from jax.experimental import pallas as pl
from jax.experimental.pallas import tpu as pltpu
from jax.experimental.pallas import tpu_sc as plsc

N = 10000
E = 320000
F = 128
H = 32
G = 64
CLS = 10

NC = 2
NS = 16
NW = NC * NS
CH = 128
K = 80
EPAD = NW * K * CH
NPAD = 10240
RZ = NPAD // NS
D = 8


def _segsum_body(p_hbm, src_hbm, dst_hbm, zeros_hbm, out_hbm, srcb, dstb, rows, acc, sem):
    c = lax.axis_index("c")
    s = lax.axis_index("s")
    wid = c * NS + s
    pltpu.sync_copy(zeros_hbm.at[pl.ds(s * RZ, RZ)], acc.at[pl.ds(s * RZ, RZ)])
    pltpu.sync_copy(src_hbm.at[pl.ds(wid * K, K)], srcb)
    pltpu.sync_copy(dst_hbm.at[pl.ds(wid * K, K)], dstb)
    plsc.subcore_barrier()

    for b in range(D):
        pltpu.async_copy(p_hbm.at[srcb.at[b]], rows.at[b], sem)

    def step(j, carry):
        jm = lax.rem(j, D)
        pltpu.make_async_copy(p_hbm.at[srcb.at[j]], rows.at[jm], sem).wait()
        pltpu.sync_copy(rows.at[jm], acc.at[dstb.at[j]], add=True)

        @pl.when(j + D < K)
        def _():
            pltpu.async_copy(p_hbm.at[srcb.at[j + D]], rows.at[jm], sem)

        return carry

    lax.fori_loop(0, K, step, 0)
    plsc.subcore_barrier()
    pltpu.sync_copy(acc.at[pl.ds(s * RZ, RZ)], out_hbm.at[c, pl.ds(s * RZ, RZ)])


@functools.lru_cache(maxsize=1)
def _make_segsum():
    return pl.kernel(
        _segsum_body,
        out_type=jax.ShapeDtypeStruct((NC, NPAD, H), jnp.float32),
        mesh=plsc.VectorSubcoreMesh(core_axis_name="c", subcore_axis_name="s"),
        scratch_types=[
            pltpu.VMEM((K, CH), jnp.int32),
            pltpu.VMEM((K, CH), jnp.int32),
            pltpu.VMEM((D, CH, H), jnp.float32),
            pltpu.VMEM_SHARED((NPAD, H), jnp.float32),
            pltpu.SemaphoreType.DMA,
        ],
        compiler_params=pltpu.CompilerParams(use_tc_tiling_on_sc=False),
    )


def _proj_body(x_ref, w_ref, o_ref):
    o_ref[...] = jnp.dot(x_ref[...], w_ref[...], preferred_element_type=jnp.float32)


_proj = pl.pallas_call(_proj_body, out_shape=jax.ShapeDtypeStruct((N, H), jnp.float32))


def _mlp_bn(p, pa, pb, b1, w2, b2, gam, bet):
    z = jnp.maximum(p + pa + pb + b1, 0.0)
    z = jnp.maximum(jnp.dot(z, w2, preferred_element_type=jnp.float32) + b2, 0.0)
    mu = jnp.mean(z, axis=0, keepdims=True)
    zc = z - mu
    var = jnp.mean(zc * zc, axis=0, keepdims=True)
    return zc * lax.rsqrt(var + 1e-5) * gam + bet


def _layer_body(p_ref, pa_ref, pb_ref, b1_ref, w2_ref, b2_ref, g_ref, be_ref,
                w1n_ref, o_ref):
    h = _mlp_bn(p_ref[...], pa_ref[...], pb_ref[...], b1_ref[...], w2_ref[...],
                b2_ref[...], g_ref[...], be_ref[...])
    o_ref[...] = jnp.dot(h, w1n_ref[...], preferred_element_type=jnp.float32)


_layer = pl.pallas_call(_layer_body, out_shape=jax.ShapeDtypeStruct((N, H), jnp.float32))


def _head_body(p_ref, pa_ref, pb_ref, b1_ref, w2_ref, b2_ref, g_ref, be_ref,
               batch_ref, fc1w_ref, fc1b_ref, fc2w_ref, fc2b_ref, o_ref):
    h = _mlp_bn(p_ref[...], pa_ref[...], pb_ref[...], b1_ref[...], w2_ref[...],
                b2_ref[...], g_ref[...], be_ref[...])
    bo = (batch_ref[...] == lax.broadcasted_iota(jnp.int32, (1, G), 1))
    bo = bo.astype(jnp.float32)
    dn = (((0,), (0,)), ((), ()))
    sums = lax.dot_general(bo, h, dn, preferred_element_type=jnp.float32)
    counts = lax.dot_general(bo, jnp.ones((N, 1), jnp.float32), dn,
                             preferred_element_type=jnp.float32)
    pooled = sums / jnp.maximum(counts, 1.0)
    z = jnp.maximum(jnp.dot(pooled, fc1w_ref[...],
                            preferred_element_type=jnp.float32) + fc1b_ref[...], 0.0)
    logits = jnp.dot(z, fc2w_ref[...], preferred_element_type=jnp.float32) + fc2b_ref[...]
    m = jnp.max(logits, axis=-1, keepdims=True)
    lse = m + jnp.log(jnp.sum(jnp.exp(logits - m), axis=-1, keepdims=True))
    o_ref[...] = logits - lse


_head = pl.pallas_call(_head_body, out_shape=jax.ShapeDtypeStruct((G, CLS), jnp.float32))


def kernel(x, params, edge_index, batch):
    ei = edge_index.astype(jnp.int32)
    pad = EPAD - E
    src2 = jnp.concatenate([ei[0], jnp.zeros((pad,), jnp.int32)]).reshape(NW * K, CH)
    dst2 = jnp.concatenate([ei[1], jnp.full((pad,), N, jnp.int32)]).reshape(NW * K, CH)
    zeros = jnp.zeros((NPAD, H), jnp.float32)
    b2d = batch.astype(jnp.int32).reshape(N, 1)
    vec = lambda v: v.reshape(1, -1)

    segsum = _make_segsum()
    p = _proj(x, params["conv1_W1"])
    for i in range(1, 6):
        parts = segsum(p, src2, dst2, zeros)
        pa = parts[0, :N]
        pb = parts[1, :N]
        if i < 5:
            p = _layer(p, pa, pb, vec(params[f"conv{i}_b1"]), params[f"conv{i}_W2"],
                       vec(params[f"conv{i}_b2"]), vec(params[f"bn{i}_gamma"]),
                       vec(params[f"bn{i}_beta"]), params[f"conv{i + 1}_W1"])
        else:
            out = _head(p, pa, pb, vec(params[f"conv{i}_b1"]), params[f"conv{i}_W2"],
                        vec(params[f"conv{i}_b2"]), vec(params[f"bn{i}_gamma"]),
                        vec(params[f"bn{i}_beta"]), b2d, params["fc1_W"],
                        vec(params["fc1_b"]), params["fc2_W"], vec(params["fc2_b"]))
    return out


# speedup vs baseline: 17.3339x; 1.9007x over previous
import functools

import jax
import jax.numpy as jnp
from jax import lax
from jax.experimental import pallas as pl
from jax.experimental.pallas import tpu as pltpu
from jax.experimental.pallas import tpu_sc as plsc

N = 10000
E = 320000
F = 128
H = 32
G = 64
CLS = 10

NC = 2
NS = 16
NW = NC * NS
CH = 128
K = 80
EPAD = NW * K * CH
NPAD = 10240
RZ = NPAD // NS
D = 8


def _segsum_body(p_hbm, src_hbm, dst_hbm, zeros_hbm, out_hbm, srcb, dstb, rows,
                 acc, pshr, sem):
    c = lax.axis_index("c")
    s = lax.axis_index("s")
    wid = c * NS + s
    pltpu.sync_copy(zeros_hbm.at[pl.ds(s * RZ, RZ)], acc.at[pl.ds(s * RZ, RZ)])
    pltpu.sync_copy(p_hbm.at[pl.ds(s * (N // NS), N // NS)],
                    pshr.at[pl.ds(s * (N // NS), N // NS)])
    pltpu.sync_copy(src_hbm.at[pl.ds(wid * K, K)], srcb)
    pltpu.sync_copy(dst_hbm.at[pl.ds(wid * K, K)], dstb)
    plsc.subcore_barrier()

    for b in range(D):
        pltpu.async_copy(pshr.at[srcb.at[b]], rows.at[b], sem)

    def step(j, carry):
        jm = lax.rem(j, D)
        pltpu.make_async_copy(pshr.at[srcb.at[j]], rows.at[jm], sem).wait()
        pltpu.sync_copy(rows.at[jm], acc.at[dstb.at[j]], add=True)

        @pl.when(j + D < K)
        def _():
            pltpu.async_copy(pshr.at[srcb.at[j + D]], rows.at[jm], sem)

        return carry

    lax.fori_loop(0, K, step, 0)
    plsc.subcore_barrier()
    pltpu.sync_copy(acc.at[pl.ds(s * RZ, RZ)], out_hbm.at[c, pl.ds(s * RZ, RZ)])


@functools.lru_cache(maxsize=1)
def _make_segsum():
    return pl.kernel(
        _segsum_body,
        out_type=jax.ShapeDtypeStruct((NC, NPAD, H), jnp.float32),
        mesh=plsc.VectorSubcoreMesh(core_axis_name="c", subcore_axis_name="s"),
        scratch_types=[
            pltpu.VMEM((K, CH), jnp.int32),
            pltpu.VMEM((K, CH), jnp.int32),
            pltpu.VMEM((D, CH, H), jnp.float32),
            pltpu.VMEM_SHARED((NPAD, H), jnp.float32),
            pltpu.VMEM_SHARED((N, H), jnp.float32),
            pltpu.SemaphoreType.DMA,
        ],
        compiler_params=pltpu.CompilerParams(use_tc_tiling_on_sc=False),
    )


def _proj_body(x_ref, w_ref, o_ref):
    o_ref[...] = jnp.dot(x_ref[...], w_ref[...], preferred_element_type=jnp.float32)


_proj = pl.pallas_call(_proj_body, out_shape=jax.ShapeDtypeStruct((N, H), jnp.float32))


def _mlp_bn(p, pa, pb, b1, w2, b2, gam, bet):
    z = jnp.maximum(p + pa + pb + b1, 0.0)
    z = jnp.maximum(jnp.dot(z, w2, preferred_element_type=jnp.float32) + b2, 0.0)
    mu = jnp.mean(z, axis=0, keepdims=True)
    zc = z - mu
    var = jnp.mean(zc * zc, axis=0, keepdims=True)
    return zc * lax.rsqrt(var + 1e-5) * gam + bet


def _layer_body(p_ref, pa_ref, pb_ref, b1_ref, w2_ref, b2_ref, g_ref, be_ref,
                w1n_ref, o_ref):
    h = _mlp_bn(p_ref[...], pa_ref[...], pb_ref[...], b1_ref[...], w2_ref[...],
                b2_ref[...], g_ref[...], be_ref[...])
    o_ref[...] = jnp.dot(h, w1n_ref[...], preferred_element_type=jnp.float32)


_layer = pl.pallas_call(_layer_body, out_shape=jax.ShapeDtypeStruct((N, H), jnp.float32))


def _head_body(p_ref, pa_ref, pb_ref, b1_ref, w2_ref, b2_ref, g_ref, be_ref,
               batch_ref, fc1w_ref, fc1b_ref, fc2w_ref, fc2b_ref, o_ref):
    h = _mlp_bn(p_ref[...], pa_ref[...], pb_ref[...], b1_ref[...], w2_ref[...],
                b2_ref[...], g_ref[...], be_ref[...])
    bo = (batch_ref[...] == lax.broadcasted_iota(jnp.int32, (1, G), 1))
    bo = bo.astype(jnp.float32)
    dn = (((0,), (0,)), ((), ()))
    sums = lax.dot_general(bo, h, dn, preferred_element_type=jnp.float32)
    counts = lax.dot_general(bo, jnp.ones((N, 1), jnp.float32), dn,
                             preferred_element_type=jnp.float32)
    pooled = sums / jnp.maximum(counts, 1.0)
    z = jnp.maximum(jnp.dot(pooled, fc1w_ref[...],
                            preferred_element_type=jnp.float32) + fc1b_ref[...], 0.0)
    logits = jnp.dot(z, fc2w_ref[...], preferred_element_type=jnp.float32) + fc2b_ref[...]
    m = jnp.max(logits, axis=-1, keepdims=True)
    lse = m + jnp.log(jnp.sum(jnp.exp(logits - m), axis=-1, keepdims=True))
    o_ref[...] = logits - lse


_head = pl.pallas_call(_head_body, out_shape=jax.ShapeDtypeStruct((G, CLS), jnp.float32))


def kernel(x, params, edge_index, batch):
    ei = edge_index.astype(jnp.int32)
    pad = EPAD - E
    src2 = jnp.concatenate([ei[0], jnp.zeros((pad,), jnp.int32)]).reshape(NW * K, CH)
    dst2 = jnp.concatenate([ei[1], jnp.full((pad,), N, jnp.int32)]).reshape(NW * K, CH)
    zeros = jnp.zeros((NPAD, H), jnp.float32)
    b2d = batch.astype(jnp.int32).reshape(N, 1)
    vec = lambda v: v.reshape(1, -1)

    segsum = _make_segsum()
    p = _proj(x, params["conv1_W1"])
    for i in range(1, 6):
        parts = segsum(p, src2, dst2, zeros)
        pa = parts[0, :N]
        pb = parts[1, :N]
        if i < 5:
            p = _layer(p, pa, pb, vec(params[f"conv{i}_b1"]), params[f"conv{i}_W2"],
                       vec(params[f"conv{i}_b2"]), vec(params[f"bn{i}_gamma"]),
                       vec(params[f"bn{i}_beta"]), params[f"conv{i + 1}_W1"])
        else:
            out = _head(p, pa, pb, vec(params[f"conv{i}_b1"]), params[f"conv{i}_W2"],
                        vec(params[f"conv{i}_b2"]), vec(params[f"bn{i}_gamma"]),
                        vec(params[f"bn{i}_beta"]), b2d, params["fc1_W"],
                        vec(params["fc1_b"]), params["fc2_W"], vec(params["fc2_b"]))
    return out


# speedup vs baseline: 25.3048x; 1.4598x over previous
import functools

import jax
import jax.numpy as jnp
from jax import lax
from jax.experimental import pallas as pl
from jax.experimental.pallas import tpu as pltpu
from jax.experimental.pallas import tpu_sc as plsc

N = 10000
E = 320000
F = 128
H = 32
G = 64
CLS = 10

NC = 2
NS = 16
NW = NC * NS
CH = 128
K = 80
EPAD = NW * K * CH
NPAD = 10240
RP = NPAD // 4
RN = N // 4
RS = RP // NS
D = 8


def _segsum_body(p_hbm, src_hbm, dst_hbm, zeros_hbm, out_hbm, srcb, dstb, rows,
                 acc, pshr, sem):
    c = lax.axis_index("c")
    s = lax.axis_index("s")
    wid = c * NS + s
    rz = NPAD // NS
    pltpu.sync_copy(zeros_hbm.at[pl.ds(s * rz, rz)], acc.at[pl.ds(s * rz, rz)])
    pltpu.sync_copy(p_hbm.at[pl.ds(s * rz, rz)], pshr.at[pl.ds(s * rz, rz)])
    pltpu.sync_copy(src_hbm.at[pl.ds(wid * K, K)], srcb)
    pltpu.sync_copy(dst_hbm.at[pl.ds(wid * K, K)], dstb)
    plsc.subcore_barrier()

    for b in range(D):
        pltpu.async_copy(pshr.at[srcb.at[b]], rows.at[b], sem)

    def step(j, carry):
        jm = lax.rem(j, D)
        pltpu.make_async_copy(pshr.at[srcb.at[j]], rows.at[jm], sem).wait()
        pltpu.sync_copy(rows.at[jm], acc.at[dstb.at[j]], add=True)

        @pl.when(j + D < K)
        def _():
            pltpu.async_copy(pshr.at[srcb.at[j + D]], rows.at[jm], sem)

        return carry

    lax.fori_loop(0, K, step, 0)
    plsc.subcore_barrier()
    pltpu.sync_copy(acc.at[pl.ds(s * rz, rz)], out_hbm.at[c, pl.ds(s * rz, rz)])


@functools.lru_cache(maxsize=1)
def _make_segsum():
    return pl.kernel(
        _segsum_body,
        out_type=jax.ShapeDtypeStruct((NC, NPAD, H), jnp.float32),
        mesh=plsc.VectorSubcoreMesh(core_axis_name="c", subcore_axis_name="s"),
        scratch_types=[
            pltpu.VMEM((K, CH), jnp.int32),
            pltpu.VMEM((K, CH), jnp.int32),
            pltpu.VMEM((D, CH, H), jnp.float32),
            pltpu.VMEM_SHARED((NPAD, H), jnp.float32),
            pltpu.VMEM_SHARED((NPAD, H), jnp.float32),
            pltpu.SemaphoreType.DMA,
        ],
        compiler_params=pltpu.CompilerParams(use_tc_tiling_on_sc=False),
    )


def _fold4(v, n):
    ri = lax.rem(lax.broadcasted_iota(jnp.int32, (F, F), 0), H)
    cj = lax.rem(lax.broadcasted_iota(jnp.int32, (F, F), 1), H)
    m = (ri == cj).astype(jnp.float32)
    return jnp.dot(v, m, preferred_element_type=jnp.float32) / n


def _mlp_bn(p, pa, pb, b1, w2big, b2, gam, bet):
    z = jnp.maximum(p + pa + pb + b1, 0.0)
    z = jnp.maximum(jnp.dot(z, w2big, preferred_element_type=jnp.float32) + b2, 0.0)
    zs = z[0:RN]
    mu = _fold4(jnp.sum(zs, axis=0, keepdims=True), float(N))
    zc = z - mu
    zcs = zc[0:RN]
    var = _fold4(jnp.sum(zcs * zcs, axis=0, keepdims=True), float(N))
    return zc * lax.rsqrt(var + 1e-5) * gam + bet


def _proj_body(x_ref, w1big_ref, o_ref):
    o_ref[0:RN, :] = jnp.dot(x_ref[...], w1big_ref[...],
                             preferred_element_type=jnp.float32)
    o_ref[RN:RP, :] = jnp.zeros((RP - RN, F), jnp.float32)


_proj = pl.pallas_call(_proj_body, out_shape=jax.ShapeDtypeStruct((RP, F), jnp.float32))


def _layer_body(p_ref, parts_ref, b1_ref, w2big_ref, b2_ref, g_ref, be_ref,
                w1nbig_ref, o_ref):
    h = _mlp_bn(p_ref[...], parts_ref[0], parts_ref[1], b1_ref[...],
                w2big_ref[...], b2_ref[...], g_ref[...], be_ref[...])
    o_ref[...] = jnp.dot(h, w1nbig_ref[...], preferred_element_type=jnp.float32)


_layer = pl.pallas_call(_layer_body, out_shape=jax.ShapeDtypeStruct((RP, F), jnp.float32))


def _head_body(p_ref, parts_ref, b1_ref, w2big_ref, b2_ref, g_ref, be_ref,
               batch_ref, fc1w_ref, fc1b_ref, fc2w_ref, fc2b_ref, o_ref):
    h = _mlp_bn(p_ref[...], parts_ref[0], parts_ref[1], b1_ref[...],
                w2big_ref[...], b2_ref[...], g_ref[...], be_ref[...])
    giota = lax.broadcasted_iota(jnp.int32, (1, G), 1)
    ones = jnp.ones((RN, 1), jnp.float32)
    dn = (((0,), (0,)), ((), ()))
    sums = jnp.zeros((G, H), jnp.float32)
    counts = jnp.zeros((G, 1), jnp.float32)
    for a in range(4):
        mask = (batch_ref[:, a:a + 1] == giota).astype(jnp.float32)
        ha = h[0:RN, a * H:(a + 1) * H]
        sums = sums + lax.dot_general(mask, ha, dn, preferred_element_type=jnp.float32)
        counts = counts + lax.dot_general(mask, ones, dn, preferred_element_type=jnp.float32)
    pooled = sums / jnp.maximum(counts, 1.0)
    z = jnp.maximum(jnp.dot(pooled, fc1w_ref[...],
                            preferred_element_type=jnp.float32) + fc1b_ref[...], 0.0)
    logits = jnp.dot(z, fc2w_ref[...], preferred_element_type=jnp.float32) + fc2b_ref[...]
    m = jnp.max(logits, axis=-1, keepdims=True)
    lse = m + jnp.log(jnp.sum(jnp.exp(logits - m), axis=-1, keepdims=True))
    o_ref[...] = logits - lse


_head = pl.pallas_call(_head_body, out_shape=jax.ShapeDtypeStruct((G, CLS), jnp.float32))


def kernel(x, params, edge_index, batch):
    ei = edge_index.astype(jnp.int32)
    pad = EPAD - E
    src2 = jnp.concatenate([ei[0], jnp.zeros((pad,), jnp.int32)]).reshape(NW * K, CH)
    dst2 = jnp.concatenate([ei[1], jnp.full((pad,), N, jnp.int32)]).reshape(NW * K, CH)
    zeros = jnp.zeros((NPAD, H), jnp.float32)
    batch_r = batch.astype(jnp.int32).reshape(RN, 4)
    x_r = x.reshape(RN, 4 * F)
    eye4 = jnp.eye(4, dtype=jnp.float32)
    big = lambda w: jnp.kron(eye4, w)
    vec4 = lambda v: jnp.tile(v, 4).reshape(1, F)

    segsum = _make_segsum()
    p = _proj(x_r, big(params["conv1_W1"]))
    for i in range(1, 6):
        parts = segsum(p.reshape(NPAD, H), src2, dst2, zeros).reshape(NC, RP, F)
        if i < 5:
            p = _layer(p, parts, vec4(params[f"conv{i}_b1"]), big(params[f"conv{i}_W2"]),
                       vec4(params[f"conv{i}_b2"]), vec4(params[f"bn{i}_gamma"]),
                       vec4(params[f"bn{i}_beta"]), big(params[f"conv{i + 1}_W1"]))
        else:
            out = _head(p, parts, vec4(params[f"conv{i}_b1"]), big(params[f"conv{i}_W2"]),
                        vec4(params[f"conv{i}_b2"]), vec4(params[f"bn{i}_gamma"]),
                        vec4(params[f"bn{i}_beta"]), batch_r, params["fc1_W"],
                        params["fc1_b"].reshape(1, H), params["fc2_W"],
                        params["fc2_b"].reshape(1, CLS))
    return out
